# baseline (speedup 1.0000x reference)
import functools

import jax
import jax.numpy as jnp
from jax import lax
from jax.experimental import pallas as pl
from jax.experimental.pallas import tpu as pltpu
from jax.experimental.pallas import tpu_sc as plsc

NC = 2
NS = 16
NW = NC * NS

_CH = 512
_SUB = 128


def _sc_gather(idx_all, emb, B, K, E):
    n_c = B // NW
    n_n = (B * K) // NW
    per_tile = 2 * n_c + n_n
    chunks = per_tile // _CH
    sub = _CH // _SUB

    mesh = plsc.VectorSubcoreMesh(core_axis_name="c", subcore_axis_name="s")

    @functools.partial(
        pl.kernel,
        mesh=mesh,
        compiler_params=pltpu.CompilerParams(use_tc_tiling_on_sc=False),
        out_type=[
            jax.ShapeDtypeStruct((B, E), jnp.float32),
            jax.ShapeDtypeStruct((B, E), jnp.float32),
            jax.ShapeDtypeStruct((B * K, E), jnp.float32),
        ],
        scratch_types=[
            pltpu.VMEM((per_tile,), jnp.int32),
            pltpu.VMEM((_CH, E), jnp.float32),
            pltpu.VMEM((_CH, E), jnp.float32),
            pltpu.SemaphoreType.DMA,
            pltpu.SemaphoreType.DMA,
            pltpu.SemaphoreType.DMA,
            pltpu.SemaphoreType.DMA,
        ],
    )
    def gather_k(idx_hbm, emb_hbm, c_out, t_out, n_out,
                 idx_v, buf0, buf1, g0, g1, o0, o1):
        wid = lax.axis_index("s") * NC + lax.axis_index("c")
        bufs = (buf0, buf1)
        gsems = (g0, g1)
        osems = (o0, o1)

        pltpu.sync_copy(idx_hbm.at[pl.ds(wid * n_c, n_c)],
                        idx_v.at[pl.ds(0, n_c)])
        pltpu.sync_copy(idx_hbm.at[pl.ds(B + wid * n_c, n_c)],
                        idx_v.at[pl.ds(n_c, n_c)])
        pltpu.sync_copy(idx_hbm.at[pl.ds(2 * B + wid * n_n, n_n)],
                        idx_v.at[pl.ds(2 * n_c, n_n)])

        def start(c, bi):
            hs = []
            for j in range(sub):
                off = c * _CH + j * _SUB
                hs.append(pltpu.async_copy(
                    emb_hbm.at[idx_v.at[pl.ds(off, _SUB)]],
                    bufs[bi].at[pl.ds(j * _SUB, _SUB)],
                    gsems[bi]))
            return hs

        def dest(c):
            if c * _CH < n_c:
                return c_out, wid * n_c + c * _CH
            if c * _CH < 2 * n_c:
                return t_out, wid * n_c + (c * _CH - n_c)
            return n_out, wid * n_n + (c * _CH - 2 * n_c)

        pend_g = {0: start(0, 0)}
        pend_o = {}
        for c in range(chunks):
            bi = c % 2
            if c + 1 < chunks:
                if (1 - bi) in pend_o:
                    pend_o.pop(1 - bi).wait()
                pend_g[1 - bi] = start(c + 1, 1 - bi)
            for h in pend_g.pop(bi):
                h.wait()
            out_ref, base = dest(c)
            pend_o[bi] = pltpu.async_copy(
                bufs[bi], out_ref.at[pl.ds(base, _CH)], osems[bi])
        for h in pend_o.values():
            h.wait()

    return gather_k(idx_all, emb)


def _transpose_body(in_ref, out_ref):
    out_ref[...] = jnp.transpose(in_ref[...])


def _tc_transpose(embT):
    E, V = embT.shape
    C = 8192
    return pl.pallas_call(
        _transpose_body,
        grid=(V // C,),
        in_specs=[pl.BlockSpec((E, C), lambda i: (0, i))],
        out_specs=pl.BlockSpec((C, E), lambda i: (i, 0)),
        out_shape=jax.ShapeDtypeStruct((V, E), jnp.float32),
    )(embT)


def _logsig(x):
    return jnp.minimum(x, 0.0) - jnp.log(1.0 + jnp.exp(-jnp.abs(x)))


def _tc_body(B, K, nb,
             gc_ref, gt_ref, gn_ref, lab_ref,
             eWf_ref, ebf_ref, dWf_ref, dbf_ref, Af_ref, cvf_ref,
             cWf_ref, cbf_ref,
             eWg_ref, ebg_ref, dWg_ref, dbg_ref, Ag_ref, cvg_ref,
             cWg_ref, cbg_ref,
             recWf_ref, recWg_ref, recb_ref,
             out_ref):
    i = pl.program_id(0)

    @pl.when(i == 0)
    def _init():
        out_ref[...] = jnp.zeros_like(out_ref)

    hi = lax.Precision.HIGHEST
    f32 = jnp.float32

    def mm(a, b):
        return jnp.dot(a, b, precision=hi, preferred_element_type=f32)

    Ct = jnp.transpose(gc_ref[...])
    Tt = jnp.transpose(gt_ref[...])
    lab = lab_ref[0]

    def decomposer(eWT, ebT, dWT, dbT, A, cvT, cWT, cbT):
        enc_c = mm(eWT, Ct) + ebT
        enc_t = mm(eWT, Tt) + ebT
        dc = mm(dWT, enc_c) + dbT
        dt = mm(dWT, enc_t) + dbT
        obj = _logsig(jnp.sum(dc * dt, axis=0, keepdims=True))
        w = mm(A, dc)
        s = jnp.sum(dc * cvT, axis=0, keepdims=True)
        rows = []
        for k in range(K):
            nk = jnp.transpose(gn_ref[k])
            rows.append(jnp.sum(nk * w, axis=0, keepdims=True))
        S = jnp.concatenate(rows, axis=0)
        nobj = jnp.sum(_logsig(-(S + s)), axis=0, keepdims=True)
        deno_sum = jnp.sum(obj + nobj)
        logits = mm(cWT, enc_c) + cbT
        l0 = logits[0:1, :]
        l1 = logits[1:2, :]
        m = jnp.maximum(l0, l1)
        lse = m + jnp.log(jnp.exp(l0 - m) + jnp.exp(l1 - m))
        pick = jnp.where(lab == 0.0, l0, l1)
        cono_sum = jnp.sum(pick - lse)
        return enc_c, deno_sum, cono_sum

    enc_f, deno_f, cono_f = decomposer(
        eWf_ref[...], ebf_ref[...], dWf_ref[...], dbf_ref[...],
        Af_ref[...], cvf_ref[...], cWf_ref[...], cbf_ref[...])
    enc_g, deno_g, cono_g = decomposer(
        eWg_ref[...], ebg_ref[...], dWg_ref[...], dbg_ref[...],
        Ag_ref[...], cvg_ref[...], cWg_ref[...], cbg_ref[...])

    rec = mm(recWf_ref[...], enc_f) + mm(recWg_ref[...], enc_g) + recb_ref[...]
    num = jnp.sum(Ct * rec, axis=0, keepdims=True)
    den = (jnp.sqrt(jnp.sum(Ct * Ct, axis=0, keepdims=True))
           * jnp.sqrt(jnp.sum(rec * rec, axis=0, keepdims=True)) + 1e-8)
    cos_sum = jnp.sum(num / den)

    out_ref[0:1, :] = out_ref[0:1, :] + deno_f
    out_ref[1:2, :] = out_ref[1:2, :] + cono_f
    out_ref[2:3, :] = out_ref[2:3, :] + deno_g
    out_ref[3:4, :] = out_ref[3:4, :] + cono_g
    out_ref[4:5, :] = out_ref[4:5, :] + cos_sum

    @pl.when(i == nb - 1)
    def _fin():
        v = out_ref[...]
        invB = 1.0 / B
        l_f_deno = -v[0:1, :] * invB
        l_f_cono = -v[1:2, :] * invB
        l_g_deno = -v[2:3, :] * invB
        l_g_cono = -v[3:4, :] * invB
        l_h = 1.0 - v[4:5, :] * invB
        L_master = l_f_deno + l_f_cono + l_g_deno + l_g_cono + l_h
        out_ref[0:1, :] = L_master
        out_ref[1:2, :] = l_f_deno
        out_ref[2:3, :] = l_f_cono
        out_ref[3:4, :] = l_g_deno
        out_ref[4:5, :] = l_g_cono
        out_ref[5:6, :] = l_h


def _tc_compute(gc, gt, gn3, labf,
                eWf, ebf, dWf, dbf, Af, cvf, cWf, cbf,
                eWg, ebg, dWg, dbg, Ag, cvg, cWg, cbg,
                recWf, recWg, recb, interpret=False):
    B, E = gc.shape
    K = gn3.shape[0]
    Bb = 512
    nb = B // Bb
    D = eWf.shape[0]

    def full(shape):
        nd = len(shape)
        return pl.BlockSpec(shape, lambda i, nd=nd: (0,) * nd)

    in_specs = [
        pl.BlockSpec((Bb, E), lambda i: (i, 0)),
        pl.BlockSpec((Bb, E), lambda i: (i, 0)),
        pl.BlockSpec((K, Bb, E), lambda i: (0, i, 0)),
        pl.BlockSpec((1, 1, Bb), lambda i: (i, 0, 0)),
        full((D, E)), full((D, 1)), full((E, D)), full((E, 1)),
        full((E, E)), full((E, 1)), full((2, D)), full((2, 1)),
        full((D, E)), full((D, 1)), full((E, D)), full((E, 1)),
        full((E, E)), full((E, 1)), full((2, D)), full((2, 1)),
        full((E, D)), full((E, D)), full((E, 1)),
    ]
    out = pl.pallas_call(
        functools.partial(_tc_body, B, K, nb),
        grid=(nb,),
        in_specs=in_specs,
        out_specs=pl.BlockSpec((8, 128), lambda i: (0, 0)),
        out_shape=jax.ShapeDtypeStruct((8, 128), jnp.float32),
        interpret=interpret,
    )(gc, gt, gn3, labf,
      eWf, ebf, dWf, dbf, Af, cvf, cWf, cbf,
      eWg, ebg, dWg, dbg, Ag, cvg, cWg, cbg,
      recWf, recWg, recb)
    return out[:6, 0]


def kernel(emb, enc_f_W, enc_f_b, f_deno_W, f_deno_b, f_cono_W, f_cono_b,
           enc_g_W, enc_g_b, g_deno_W, g_deno_b, g_cono_W, g_cono_b,
           rec_W, rec_b,
           center_word_ids, context_word_ids, negative_context_ids,
           party_labels):
    B = center_word_ids.shape[0]
    K = negative_context_ids.shape[1]
    E = emb.shape[1]
    D = enc_f_W.shape[1]

    i32 = jnp.int32
    idx_all = jnp.concatenate([
        center_word_ids.astype(i32),
        context_word_ids.astype(i32),
        negative_context_ids.astype(i32).T.reshape(-1),
    ])

    emb_rm = _tc_transpose(jnp.transpose(emb))
    gc, gt, gn = _sc_gather(idx_all, emb_rm, B, K, E)
    gn3 = gn.reshape(K, B, E)

    labf = party_labels.astype(jnp.float32).reshape(B // 512, 1, 512)
    Af = enc_f_W @ f_deno_W
    cvf = (enc_f_b @ f_deno_W + f_deno_b).reshape(E, 1)
    Ag = enc_g_W @ g_deno_W
    cvg = (enc_g_b @ g_deno_W + g_deno_b).reshape(E, 1)

    return _tc_compute(
        gc, gt, gn3, labf,
        enc_f_W.T, enc_f_b.reshape(D, 1), f_deno_W.T, f_deno_b.reshape(E, 1),
        Af, cvf, f_cono_W.T, f_cono_b.reshape(2, 1),
        enc_g_W.T, enc_g_b.reshape(D, 1), g_deno_W.T, g_deno_b.reshape(E, 1),
        Ag, cvg, g_cono_W.T, g_cono_b.reshape(2, 1),
        rec_W[:D].T, rec_W[D:].T, rec_b.reshape(E, 1))


# speedup vs baseline: 1.5167x; 1.5167x over previous
import functools

import jax
import jax.numpy as jnp
from jax import lax
from jax.experimental import pallas as pl
from jax.experimental.pallas import tpu as pltpu
from jax.experimental.pallas import tpu_sc as plsc

NC = 2
NS = 16
NW = NC * NS

_CH = 256
_SUB = 128


def _sc_gather(idx_all, emb, B, K, E):
    n_c = B // NW
    n_n = (B * K) // NW
    per_tile = 2 * n_c + n_n
    chunks = per_tile // _CH
    sub = _CH // _SUB

    mesh = plsc.VectorSubcoreMesh(core_axis_name="c", subcore_axis_name="s")

    @functools.partial(
        pl.kernel,
        mesh=mesh,
        compiler_params=pltpu.CompilerParams(use_tc_tiling_on_sc=False),
        out_type=[
            jax.ShapeDtypeStruct((B, E), jnp.float32),
            jax.ShapeDtypeStruct((B, E), jnp.float32),
            jax.ShapeDtypeStruct((B * K, E), jnp.float32),
        ],
        scratch_types=[
            pltpu.VMEM((per_tile,), jnp.int32),
            pltpu.VMEM((_CH, 128), jnp.float32),
            pltpu.VMEM((_CH, 128), jnp.float32),
            pltpu.SemaphoreType.DMA,
            pltpu.SemaphoreType.DMA,
            pltpu.SemaphoreType.DMA,
            pltpu.SemaphoreType.DMA,
        ],
    )
    def gather_k(idx_hbm, emb_hbm, c_out, t_out, n_out,
                 idx_v, buf0, buf1, g0, g1, o0, o1):
        wid = lax.axis_index("s") * NC + lax.axis_index("c")
        bufs = (buf0, buf1)
        gsems = (g0, g1)
        osems = (o0, o1)

        pltpu.sync_copy(idx_hbm.at[pl.ds(wid * n_c, n_c)],
                        idx_v.at[pl.ds(0, n_c)])
        pltpu.sync_copy(idx_hbm.at[pl.ds(B + wid * n_c, n_c)],
                        idx_v.at[pl.ds(n_c, n_c)])
        pltpu.sync_copy(idx_hbm.at[pl.ds(2 * B + wid * n_n, n_n)],
                        idx_v.at[pl.ds(2 * n_c, n_n)])

        def start(c, bi):
            hs = []
            for j in range(sub):
                off = c * _CH + j * _SUB
                hs.append(pltpu.async_copy(
                    emb_hbm.at[idx_v.at[pl.ds(off, _SUB)]],
                    bufs[bi].at[pl.ds(j * _SUB, _SUB)],
                    gsems[bi]))
            return hs

        def dest(c):
            if c * _CH < n_c:
                return c_out, wid * n_c + c * _CH
            if c * _CH < 2 * n_c:
                return t_out, wid * n_c + (c * _CH - n_c)
            return n_out, wid * n_n + (c * _CH - 2 * n_c)

        pend_g = {0: start(0, 0)}
        pend_o = {}
        for c in range(chunks):
            bi = c % 2
            if c + 1 < chunks:
                if (1 - bi) in pend_o:
                    pend_o.pop(1 - bi).wait()
                pend_g[1 - bi] = start(c + 1, 1 - bi)
            for h in pend_g.pop(bi):
                h.wait()
            out_ref, base = dest(c)
            pend_o[bi] = pltpu.async_copy(
                bufs[bi].at[:, pl.ds(0, E)],
                out_ref.at[pl.ds(base, _CH)], osems[bi])
        for h in pend_o.values():
            h.wait()

    return gather_k(idx_all, emb)


def _transpose_body(in_ref, out_ref):
    x = in_ref[...]
    out_ref[:, 0 : x.shape[0]] = jnp.transpose(x)


def _tc_transpose(embT):
    E, V = embT.shape
    C = 8192
    return pl.pallas_call(
        _transpose_body,
        grid=(V // C,),
        in_specs=[pl.BlockSpec((E, C), lambda i: (0, i))],
        out_specs=pl.BlockSpec((C, 128), lambda i: (i, 0)),
        out_shape=jax.ShapeDtypeStruct((V, 128), jnp.float32),
    )(embT)


def _logsig(x):
    return jnp.minimum(x, 0.0) - jnp.log(1.0 + jnp.exp(-jnp.abs(x)))


def _tc_body(B, K, nb,
             gc_ref, gt_ref, gn_ref, lab_ref,
             eWf_ref, ebf_ref, dWf_ref, dbf_ref, Af_ref, cvf_ref,
             cWf_ref, cbf_ref,
             eWg_ref, ebg_ref, dWg_ref, dbg_ref, Ag_ref, cvg_ref,
             cWg_ref, cbg_ref,
             recWf_ref, recWg_ref, recb_ref,
             out_ref):
    i = pl.program_id(0)

    @pl.when(i == 0)
    def _init():
        out_ref[...] = jnp.zeros_like(out_ref)

    hi = lax.Precision.HIGHEST
    f32 = jnp.float32

    def mm(a, b):
        return jnp.dot(a, b, precision=hi, preferred_element_type=f32)

    Ct = jnp.transpose(gc_ref[...])
    Tt = jnp.transpose(gt_ref[...])
    lab = lab_ref[0]

    def decomposer(eWT, ebT, dWT, dbT, A, cvT, cWT, cbT):
        enc_c = mm(eWT, Ct) + ebT
        enc_t = mm(eWT, Tt) + ebT
        dc = mm(dWT, enc_c) + dbT
        dt = mm(dWT, enc_t) + dbT
        obj = _logsig(jnp.sum(dc * dt, axis=0, keepdims=True))
        w = mm(A, dc)
        s = jnp.sum(dc * cvT, axis=0, keepdims=True)
        rows = []
        for k in range(K):
            nk = jnp.transpose(gn_ref[k])
            rows.append(jnp.sum(nk * w, axis=0, keepdims=True))
        S = jnp.concatenate(rows, axis=0)
        nobj = jnp.sum(_logsig(-(S + s)), axis=0, keepdims=True)
        deno_sum = jnp.sum(obj + nobj)
        logits = mm(cWT, enc_c) + cbT
        l0 = logits[0:1, :]
        l1 = logits[1:2, :]
        m = jnp.maximum(l0, l1)
        lse = m + jnp.log(jnp.exp(l0 - m) + jnp.exp(l1 - m))
        pick = jnp.where(lab == 0.0, l0, l1)
        cono_sum = jnp.sum(pick - lse)
        return enc_c, deno_sum, cono_sum

    enc_f, deno_f, cono_f = decomposer(
        eWf_ref[...], ebf_ref[...], dWf_ref[...], dbf_ref[...],
        Af_ref[...], cvf_ref[...], cWf_ref[...], cbf_ref[...])
    enc_g, deno_g, cono_g = decomposer(
        eWg_ref[...], ebg_ref[...], dWg_ref[...], dbg_ref[...],
        Ag_ref[...], cvg_ref[...], cWg_ref[...], cbg_ref[...])

    rec = mm(recWf_ref[...], enc_f) + mm(recWg_ref[...], enc_g) + recb_ref[...]
    num = jnp.sum(Ct * rec, axis=0, keepdims=True)
    den = (jnp.sqrt(jnp.sum(Ct * Ct, axis=0, keepdims=True))
           * jnp.sqrt(jnp.sum(rec * rec, axis=0, keepdims=True)) + 1e-8)
    cos_sum = jnp.sum(num / den)

    out_ref[0:1, :] = out_ref[0:1, :] + deno_f
    out_ref[1:2, :] = out_ref[1:2, :] + cono_f
    out_ref[2:3, :] = out_ref[2:3, :] + deno_g
    out_ref[3:4, :] = out_ref[3:4, :] + cono_g
    out_ref[4:5, :] = out_ref[4:5, :] + cos_sum

    @pl.when(i == nb - 1)
    def _fin():
        v = out_ref[...]
        invB = 1.0 / B
        l_f_deno = -v[0:1, :] * invB
        l_f_cono = -v[1:2, :] * invB
        l_g_deno = -v[2:3, :] * invB
        l_g_cono = -v[3:4, :] * invB
        l_h = 1.0 - v[4:5, :] * invB
        L_master = l_f_deno + l_f_cono + l_g_deno + l_g_cono + l_h
        out_ref[0:1, :] = L_master
        out_ref[1:2, :] = l_f_deno
        out_ref[2:3, :] = l_f_cono
        out_ref[3:4, :] = l_g_deno
        out_ref[4:5, :] = l_g_cono
        out_ref[5:6, :] = l_h


def _tc_compute(gc, gt, gn3, labf,
                eWf, ebf, dWf, dbf, Af, cvf, cWf, cbf,
                eWg, ebg, dWg, dbg, Ag, cvg, cWg, cbg,
                recWf, recWg, recb, interpret=False):
    B, E = gc.shape
    K = gn3.shape[0]
    Bb = 512
    nb = B // Bb
    D = eWf.shape[0]

    def full(shape):
        nd = len(shape)
        return pl.BlockSpec(shape, lambda i, nd=nd: (0,) * nd)

    in_specs = [
        pl.BlockSpec((Bb, E), lambda i: (i, 0)),
        pl.BlockSpec((Bb, E), lambda i: (i, 0)),
        pl.BlockSpec((K, Bb, E), lambda i: (0, i, 0)),
        pl.BlockSpec((1, 1, Bb), lambda i: (i, 0, 0)),
        full((D, E)), full((D, 1)), full((E, D)), full((E, 1)),
        full((E, E)), full((E, 1)), full((2, D)), full((2, 1)),
        full((D, E)), full((D, 1)), full((E, D)), full((E, 1)),
        full((E, E)), full((E, 1)), full((2, D)), full((2, 1)),
        full((E, D)), full((E, D)), full((E, 1)),
    ]
    out = pl.pallas_call(
        functools.partial(_tc_body, B, K, nb),
        grid=(nb,),
        in_specs=in_specs,
        out_specs=pl.BlockSpec((8, 128), lambda i: (0, 0)),
        out_shape=jax.ShapeDtypeStruct((8, 128), jnp.float32),
        interpret=interpret,
    )(gc, gt, gn3, labf,
      eWf, ebf, dWf, dbf, Af, cvf, cWf, cbf,
      eWg, ebg, dWg, dbg, Ag, cvg, cWg, cbg,
      recWf, recWg, recb)
    return out[:6, 0]


def kernel(emb, enc_f_W, enc_f_b, f_deno_W, f_deno_b, f_cono_W, f_cono_b,
           enc_g_W, enc_g_b, g_deno_W, g_deno_b, g_cono_W, g_cono_b,
           rec_W, rec_b,
           center_word_ids, context_word_ids, negative_context_ids,
           party_labels):
    B = center_word_ids.shape[0]
    K = negative_context_ids.shape[1]
    E = emb.shape[1]
    D = enc_f_W.shape[1]

    i32 = jnp.int32
    idx_all = jnp.concatenate([
        center_word_ids.astype(i32),
        context_word_ids.astype(i32),
        negative_context_ids.astype(i32).T.reshape(-1),
    ])

    emb_rm = _tc_transpose(jnp.transpose(emb))
    gc, gt, gn = _sc_gather(idx_all, emb_rm, B, K, E)
    gn3 = gn.reshape(K, B, E)

    labf = party_labels.astype(jnp.float32).reshape(B // 512, 1, 512)
    Af = enc_f_W @ f_deno_W
    cvf = (enc_f_b @ f_deno_W + f_deno_b).reshape(E, 1)
    Ag = enc_g_W @ g_deno_W
    cvg = (enc_g_b @ g_deno_W + g_deno_b).reshape(E, 1)

    return _tc_compute(
        gc, gt, gn3, labf,
        enc_f_W.T, enc_f_b.reshape(D, 1), f_deno_W.T, f_deno_b.reshape(E, 1),
        Af, cvf, f_cono_W.T, f_cono_b.reshape(2, 1),
        enc_g_W.T, enc_g_b.reshape(D, 1), g_deno_W.T, g_deno_b.reshape(E, 1),
        Ag, cvg, g_cono_W.T, g_cono_b.reshape(2, 1),
        rec_W[:D].T, rec_W[D:].T, rec_b.reshape(E, 1))
